# baseline (speedup 1.0000x reference)
import functools

import jax
import jax.numpy as jnp
from jax import lax
from jax.experimental import pallas as pl
from jax.experimental.pallas import tpu as pltpu
from jax.experimental.pallas import tpu_sc as plsc

F = 10
_CHUNK = 16
_RING = 2


def _leaky(x):
    return jnp.where(x >= 0, x, 0.01 * x)


def _sc_mean_body(x2_hbm, m2_hbm, *refs, node0, npw):
    bufs = refs[:_RING]
    obufs = refs[_RING:2 * _RING]
    sems = refs[2 * _RING:3 * _RING]
    osems = refs[3 * _RING:4 * _RING]
    nc = plsc.get_sparse_core_info().num_cores
    wid = lax.axis_index("s") * nc + lax.axis_index("c")
    nstart = jnp.minimum(wid * npw, m2_hbm.shape[0] - npw)
    nchunks = npw // _CHUNK
    ngroups = nchunks // _RING

    def in_copy(g, slot):
        rbase = (node0 + nstart + g * _CHUNK) * F
        return pltpu.make_async_copy(
            x2_hbm.at[pl.ds(rbase, _CHUNK * F)], bufs[slot], sems[slot])

    def out_copy(g, slot):
        nbase = nstart + g * _CHUNK
        return pltpu.make_async_copy(
            obufs[slot], m2_hbm.at[pl.ds(nbase, _CHUNK)], osems[slot])

    for b in range(_RING):
        in_copy(b, b).start()

    def group(t, _):
        for b in range(_RING):
            g = t * _RING + b
            in_copy(g, b).wait()

            @pl.when(t >= 1)
            def _():
                out_copy(g - _RING, b).wait()

            buf, ob = bufs[b], obufs[b]

            def node(n, _):
                for col in range(8):
                    c = pl.ds(col * 16, 16)
                    acc = buf[n * F, c]
                    for j in range(1, F):
                        acc = acc + buf[n * F + j, c]
                    ob[n, c] = acc * (1.0 / F)
                return 0
            lax.fori_loop(0, _CHUNK, node, 0)

            out_copy(g, b).start()

            @pl.when(g + _RING < nchunks)
            def _():
                in_copy(g + _RING, b).start()
        return 0

    lax.fori_loop(0, ngroups, group, 0)
    for b in range(_RING):
        out_copy(nchunks - _RING + b, b).wait()


def _sc_mean(x2, node0, n_nodes):
    step = _CHUNK * _RING
    npw = (-(-n_nodes // 32) + step - 1) // step * step
    mesh = plsc.VectorSubcoreMesh(core_axis_name="c", subcore_axis_name="s")
    fn = pl.kernel(
        functools.partial(_sc_mean_body, node0=node0, npw=npw),
        mesh=mesh,
        out_type=jax.ShapeDtypeStruct((n_nodes, x2.shape[1]), jnp.float32),
        scratch_types=(
            [pltpu.VMEM((_CHUNK * F, 128), jnp.float32)] * _RING
            + [pltpu.VMEM((_CHUNK, 128), jnp.float32)] * _RING
            + [pltpu.SemaphoreType.DMA] * (2 * _RING)
        ),
    )
    return fn(x2)


def _tc_m2_body(x0_r, x1g_r, m2g_r, wa0_r, ba0_r, ws0_r, wa1_r, ba1_r, ws1_r,
                out_r, *, d_in):
    ws0 = ws0_r[...]
    wa0 = wa0_r[...]
    ba0 = ba0_r[...]

    acc_h1 = None
    acc_m1 = None
    for k in range(F):
        m2k = m2g_r[:, k * d_in:(k + 1) * d_in]
        x1k = x1g_r[:, k * d_in:(k + 1) * d_in]
        h1k = _leaky(jnp.dot(x1k, ws0, preferred_element_type=jnp.float32)
                     + jnp.dot(m2k, wa0, preferred_element_type=jnp.float32)
                     + ba0)
        acc_h1 = h1k if acc_h1 is None else acc_h1 + h1k
        acc_m1 = x1k if acc_m1 is None else acc_m1 + x1k
    mh1 = acc_h1 * (1.0 / F)
    m1 = acc_m1 * (1.0 / F)

    h0 = _leaky(jnp.dot(x0_r[...], ws0, preferred_element_type=jnp.float32)
                + jnp.dot(m1, wa0, preferred_element_type=jnp.float32) + ba0)

    out_r[...] = (jnp.dot(h0, ws1_r[...], preferred_element_type=jnp.float32)
                  + jnp.dot(mh1, wa1_r[...], preferred_element_type=jnp.float32)
                  + ba1_r[...])


def kernel(x0, x1, x2, W_agg0, b_agg0, W_self0, W_agg1, b_agg1, W_self1):
    B, D_in = x0.shape
    D_h = W_agg0.shape[1]
    S = 200
    nb = B // S

    m2 = _sc_mean(x2, 0, B * F)

    x1g = x1.reshape(B, F * D_in)
    m2g = m2.reshape(B, F * D_in)
    ba0 = b_agg0.reshape(1, D_h)
    ba1 = b_agg1.reshape(1, D_h)
    w_in = [
        pl.BlockSpec((D_in, D_h), lambda i: (0, 0)),
        pl.BlockSpec((1, D_h), lambda i: (0, 0)),
        pl.BlockSpec((D_in, D_h), lambda i: (0, 0)),
        pl.BlockSpec((D_h, D_h), lambda i: (0, 0)),
        pl.BlockSpec((1, D_h), lambda i: (0, 0)),
        pl.BlockSpec((D_h, D_h), lambda i: (0, 0)),
    ]
    weights = (W_agg0, ba0, W_self0, W_agg1, ba1, W_self1)

    out = pl.pallas_call(
        functools.partial(_tc_m2_body, d_in=D_in),
        grid=(nb,),
        in_specs=[
            pl.BlockSpec((S, D_in), lambda i: (i, 0)),
            pl.BlockSpec((S, F * D_in), lambda i: (i, 0)),
            pl.BlockSpec((S, F * D_in), lambda i: (i, 0)),
        ] + w_in,
        out_specs=pl.BlockSpec((S, D_h), lambda i: (i, 0)),
        out_shape=jax.ShapeDtypeStruct((B, D_h), jnp.float32),
    )(x0, x1g, m2g, *weights)
    return out


# speedup vs baseline: 1.0351x; 1.0351x over previous
import functools

import jax
import jax.numpy as jnp
from jax import lax
from jax.experimental import pallas as pl
from jax.experimental.pallas import tpu as pltpu
from jax.experimental.pallas import tpu_sc as plsc

F = 10
_CHUNK = 16
_RING = 2


def _leaky(x):
    return jnp.where(x >= 0, x, 0.01 * x)


def _sc_mean_body(x2_hbm, m2_hbm, *refs, node0, npw):
    bufs = refs[:_RING]
    obufs = refs[_RING:2 * _RING]
    sems = refs[2 * _RING:3 * _RING]
    osems = refs[3 * _RING:4 * _RING]
    nc = plsc.get_sparse_core_info().num_cores
    wid = lax.axis_index("s") * nc + lax.axis_index("c")
    nstart = jnp.minimum(wid * npw, m2_hbm.shape[0] - npw)
    nchunks = npw // _CHUNK
    ngroups = nchunks // _RING

    def in_copy(g, slot):
        rbase = (node0 + nstart + g * _CHUNK) * F
        return pltpu.make_async_copy(
            x2_hbm.at[pl.ds(rbase, _CHUNK * F)], bufs[slot], sems[slot])

    def out_copy(g, slot):
        nbase = nstart + g * _CHUNK
        return pltpu.make_async_copy(
            obufs[slot], m2_hbm.at[pl.ds(nbase, _CHUNK)], osems[slot])

    for b in range(_RING):
        in_copy(b, b).start()

    def group(t, _):
        for b in range(_RING):
            g = t * _RING + b
            in_copy(g, b).wait()

            @pl.when(t >= 1)
            def _():
                out_copy(g - _RING, b).wait()

            buf, ob = bufs[b], obufs[b]

            def node(n, _):
                for col in range(8):
                    c = pl.ds(col * 16, 16)
                    acc = buf[n * F, c]
                    for j in range(1, F):
                        acc = acc + buf[n * F + j, c]
                    ob[n, c] = acc * (1.0 / F)
                return 0
            lax.fori_loop(0, _CHUNK, node, 0)

            out_copy(g, b).start()

            @pl.when(g + _RING < nchunks)
            def _():
                in_copy(g + _RING, b).start()
        return 0

    lax.fori_loop(0, ngroups, group, 0)
    for b in range(_RING):
        out_copy(nchunks - _RING + b, b).wait()


def _sc_mean(x2, node0, n_nodes):
    step = _CHUNK * _RING
    npw = (-(-n_nodes // 32) + step - 1) // step * step
    mesh = plsc.VectorSubcoreMesh(core_axis_name="c", subcore_axis_name="s")
    fn = pl.kernel(
        functools.partial(_sc_mean_body, node0=node0, npw=npw),
        mesh=mesh,
        out_type=jax.ShapeDtypeStruct((n_nodes, x2.shape[1]), jnp.float32),
        scratch_types=(
            [pltpu.VMEM((_CHUNK * F, 128), jnp.float32)] * _RING
            + [pltpu.VMEM((_CHUNK, 128), jnp.float32)] * _RING
            + [pltpu.SemaphoreType.DMA] * (2 * _RING)
        ),
    )
    return fn(x2)


def _tc_m2_body(x0_r, x1g_r, m2g_r, wa0_r, ba0_r, ws0_r, wa1_r, ba1_r, ws1_r,
                out_r, *, d_in):
    ws0 = ws0_r[...]
    wa0 = wa0_r[...]
    ba0 = ba0_r[...]

    acc_h1 = None
    acc_m1 = None
    for k in range(F):
        m2k = m2g_r[:, k * d_in:(k + 1) * d_in]
        x1k = x1g_r[:, k * d_in:(k + 1) * d_in]
        h1k = _leaky(jnp.dot(x1k, ws0, preferred_element_type=jnp.float32)
                     + jnp.dot(m2k, wa0, preferred_element_type=jnp.float32)
                     + ba0)
        acc_h1 = h1k if acc_h1 is None else acc_h1 + h1k
        acc_m1 = x1k if acc_m1 is None else acc_m1 + x1k
    mh1 = acc_h1 * (1.0 / F)
    m1 = acc_m1 * (1.0 / F)

    h0 = _leaky(jnp.dot(x0_r[...], ws0, preferred_element_type=jnp.float32)
                + jnp.dot(m1, wa0, preferred_element_type=jnp.float32) + ba0)

    out_r[...] = (jnp.dot(h0, ws1_r[...], preferred_element_type=jnp.float32)
                  + jnp.dot(mh1, wa1_r[...], preferred_element_type=jnp.float32)
                  + ba1_r[...])


def kernel(x0, x1, x2, W_agg0, b_agg0, W_self0, W_agg1, b_agg1, W_self1):
    B, D_in = x0.shape
    D_h = W_agg0.shape[1]
    S = 1000
    nb = B // S

    m2 = _sc_mean(x2, 0, B * F)

    x1g = x1.reshape(B, F * D_in)
    m2g = m2.reshape(B, F * D_in)
    ba0 = b_agg0.reshape(1, D_h)
    ba1 = b_agg1.reshape(1, D_h)
    w_in = [
        pl.BlockSpec((D_in, D_h), lambda i: (0, 0)),
        pl.BlockSpec((1, D_h), lambda i: (0, 0)),
        pl.BlockSpec((D_in, D_h), lambda i: (0, 0)),
        pl.BlockSpec((D_h, D_h), lambda i: (0, 0)),
        pl.BlockSpec((1, D_h), lambda i: (0, 0)),
        pl.BlockSpec((D_h, D_h), lambda i: (0, 0)),
    ]
    weights = (W_agg0, ba0, W_self0, W_agg1, ba1, W_self1)

    out = pl.pallas_call(
        functools.partial(_tc_m2_body, d_in=D_in),
        grid=(nb,),
        in_specs=[
            pl.BlockSpec((S, D_in), lambda i: (i, 0)),
            pl.BlockSpec((S, F * D_in), lambda i: (i, 0)),
            pl.BlockSpec((S, F * D_in), lambda i: (i, 0)),
        ] + w_in,
        out_specs=pl.BlockSpec((S, D_h), lambda i: (i, 0)),
        out_shape=jax.ShapeDtypeStruct((B, D_h), jnp.float32),
    )(x0, x1g, m2g, *weights)
    return out
